# baseline (speedup 1.0000x reference)
import jax
import jax.numpy as jnp
from jax import lax
from jax.experimental import pallas as pl

D = 64
B = 4096
T = 200
T0 = 10
B1 = 4096
GT = T // T0
GB = B // B1


def _body(idx_ref, x_ref, o_ref):
    idx = idx_ref[0, :]
    cols = lax.broadcasted_iota(jnp.int32, (D, D), 1)
    m = (cols == idx[:, None]).astype(jnp.bfloat16)
    for t in range(T0):
        xb = x_ref[t]
        hi = xb.astype(jnp.bfloat16)
        lo = (xb - hi.astype(jnp.float32)).astype(jnp.bfloat16)
        o_ref[t] = (jax.lax.dot(m, hi, preferred_element_type=jnp.float32)
                    + jax.lax.dot(m, lo, preferred_element_type=jnp.float32))


def _permute(xt, perm):
    return pl.pallas_call(
        _body,
        grid=(GT, GB),
        in_specs=[
            pl.BlockSpec((1, D), lambda i, k: (0, 0)),
            pl.BlockSpec((T0, D, B1), lambda i, k: (i, 0, k)),
        ],
        out_specs=pl.BlockSpec((T0, D, B1), lambda i, k: (i, 0, k)),
        out_shape=jax.ShapeDtypeStruct((T, D, B), jnp.float32),
    )(perm.reshape(1, D), xt)


def kernel(x, permutation):
    xt = jnp.transpose(x, (1, 2, 0))
    yt = _permute(xt, permutation)
    y = jnp.transpose(yt, (2, 0, 1))
    jac = jnp.zeros((x.shape[0],), dtype=x.dtype)
    return (y, jac)
